# baseline (speedup 1.0000x reference)
import functools

import jax
import jax.numpy as jnp
from jax import lax
from jax.experimental import pallas as pl
from jax.experimental.pallas import tpu as pltpu
from jax.experimental.pallas import tpu_sc as plsc

_V = 100000
_VP = 782 * 128
_E = 64
_B = 1024
_S = 200
_N = _B * _S

_PASSES = 2
_SG = _S // 8


def _pe_table_t():
    pos = jnp.arange(_S, dtype=jnp.float32)[:, None]
    denom = 10000.0 ** ((jnp.arange(_E) // 2).astype(jnp.float32) / _E)[None, :]
    ang = pos / denom
    pe = jnp.where((jnp.arange(_E) % 2)[None, :] == 0, jnp.sin(ang), jnp.cos(ang))
    return jnp.pad(pe.T, ((0, 0), (0, 1024 - _S))).reshape(_E, 8, 128)


def _sc_call(idx_flat, tp3, pe8):
    mesh = plsc.VectorSubcoreMesh(core_axis_name="c", subcore_axis_name="s")

    @functools.partial(
        pl.kernel,
        mesh=mesh,
        out_type=jax.ShapeDtypeStruct((_S, 8, 8, 8, 128), jnp.float32),
        compiler_params=pltpu.CompilerParams(needs_layout_passes=False),
        scratch_types=[
            pltpu.VMEM((782, 128), jnp.float32),
            pltpu.VMEM((8, 4, 128), jnp.int32),
            pltpu.VMEM((8, 4, 128), jnp.int32),
            pltpu.VMEM((8, 128), jnp.float32),
            pltpu.VMEM((4, 8, 128), jnp.float32),
            pltpu.VMEM((4, 8, 128), jnp.float32),
            pltpu.SemaphoreType.DMA,
            pltpu.SemaphoreType.DMA,
            pltpu.SemaphoreType.DMA,
            pltpu.SemaphoreType.DMA,
        ],
    )
    def k(idx_hbm, tp3_hbm, pe8_hbm, out_hbm, row_v, iv0, iv1, pe_v,
          ow0, ow1, ws0, ws1, is0, is1):
        sid = lax.axis_index("s")
        cid = lax.axis_index("c")
        grp_l = sid // 8
        sub = sid % 8
        obufs = (ow0, ow1)
        wsems = (ws0, ws1)
        ivs = (iv0, iv1)
        isems = (is0, is1)

        def load_unit(st, hh, iv, isem):
            s0 = st * 8 + hh * 4
            handles = []
            for j in range(8):
                handles.append(pltpu.async_copy(
                    idx_hbm.at[pl.ds(s0, 4), pl.ds(j * 128, 128)],
                    iv.at[j], isem))
            return handles

        def wait_unit(st, hh, iv, isem):
            for j in range(8):
                pltpu.make_async_copy(
                    idx_hbm.at[pl.ds(0, 4), pl.ds(j * 128, 128)],
                    iv.at[j], isem).wait()

        for p in range(_PASSES):
            g = 4 * p + 2 * cid + grp_l
            e = 8 * g + sub
            pltpu.sync_copy(tp3_hbm.at[e], row_v)
            pltpu.sync_copy(pe8_hbm.at[e], pe_v)
            load_unit(0, 0, iv0, is0)
            load_unit(jnp.int32(0), 1, iv1, is1)

            def half(st, hh, first_round):
                iv = ivs[hh]
                isem = isems[hh]
                ob = obufs[hh]
                wsem = wsems[hh]
                dst = out_hbm.at[pl.ds(st * 8 + hh * 4, 4), g, :, sub, :]
                wait_unit(st, hh, iv, isem)
                if first_round:
                    @pl.when(st >= 1)
                    def _w():
                        pltpu.make_async_copy(ob, dst, wsem).wait()
                else:
                    pltpu.make_async_copy(ob, dst, wsem).wait()
                zero16 = jnp.zeros((16,), dtype=jnp.int32)
                for si in range(4):
                    s = st * 8 + hh * 4 + si
                    hi16 = jnp.full((16,), s // 128, dtype=jnp.int32)
                    lo16 = jnp.full((16,), s % 128, dtype=jnp.int32)
                    p0 = plsc.load_gather(pe_v, [hi16, lo16])

                    @plsc.parallel_loop(0, 64, 1, unroll=64)
                    def _chunk(k, si=si, p0=p0, ob=ob, iv=iv):
                        tok = iv[k >> 3, si, pl.ds((k & 7) * 16, 16)]
                        vals = plsc.load_gather(row_v, [zero16, tok])
                        ob[si, k >> 3, pl.ds((k & 7) * 16, 16)] = vals + p0
                pltpu.async_copy(ob, dst, wsem)

                @pl.when(st + 1 < _SG)
                def _pf():
                    load_unit(st + 1, hh, iv, isem)

            def st_body(st, carry):
                half(st, 0, p == 0)
                half(st, 1, p == 0)
                return carry

            lax.fori_loop(0, _SG, st_body, 0)

        dummy = out_hbm.at[pl.ds(0, 4), 0, :, 0, :]
        pltpu.make_async_copy(obufs[0], dummy, wsems[0]).wait()
        pltpu.make_async_copy(obufs[1], dummy, wsems[1]).wait()

    return k(idx_flat, tp3, pe8)


def kernel(input_x, table):
    idx_t = input_x.T.astype(jnp.int32)
    tp3 = jnp.pad(table, ((0, _VP - _V), (0, 0))).T.reshape(_E, 782, 128)
    pe8 = _pe_table_t()
    out5 = _sc_call(idx_t, tp3, pe8)
    return out5.transpose(2, 4, 0, 1, 3).reshape(_B, _S, _E)


# speedup vs baseline: 1.4332x; 1.4332x over previous
import functools

import jax
import jax.numpy as jnp
from jax import lax
from jax.experimental import pallas as pl
from jax.experimental.pallas import tpu as pltpu
from jax.experimental.pallas import tpu_sc as plsc

_V = 100000
_VP = 782 * 128
_E = 64
_B = 1024
_S = 200
_N = _B * _S

_PASSES = 2
_SG = _S // 8


def _pe_table_t():
    pos = jnp.arange(_S, dtype=jnp.float32)[:, None]
    denom = 10000.0 ** ((jnp.arange(_E) // 2).astype(jnp.float32) / _E)[None, :]
    ang = pos / denom
    pe = jnp.where((jnp.arange(_E) % 2)[None, :] == 0, jnp.sin(ang), jnp.cos(ang))
    return jnp.pad(pe.T, ((0, 0), (0, 1024 - _S))).reshape(_E, 8, 128)


def _sc_call(idx_flat, tp3, pe8):
    mesh = plsc.VectorSubcoreMesh(core_axis_name="c", subcore_axis_name="s")

    @functools.partial(
        pl.kernel,
        mesh=mesh,
        out_type=jax.ShapeDtypeStruct((_S, 8, 8, 8, 128), jnp.float32),
        compiler_params=pltpu.CompilerParams(needs_layout_passes=False),
        scratch_types=[
            pltpu.VMEM_SHARED((_SG, 8, 8, 128), jnp.int32),
            pltpu.VMEM((782, 128), jnp.float32),
            pltpu.VMEM((8, 4, 128), jnp.int32),
            pltpu.VMEM((8, 4, 128), jnp.int32),
            pltpu.VMEM((8, 128), jnp.float32),
            pltpu.VMEM((4, 8, 128), jnp.float32),
            pltpu.VMEM((4, 8, 128), jnp.float32),
            pltpu.SemaphoreType.DMA,
            pltpu.SemaphoreType.DMA,
            pltpu.SemaphoreType.DMA,
            pltpu.SemaphoreType.DMA,
        ],
    )
    def k(idx_hbm, tp3_hbm, pe8_hbm, out_hbm, idx_sp, row_v, iv0, iv1, pe_v,
          ow0, ow1, ws0, ws1, is0, is1):
        sid = lax.axis_index("s")
        cid = lax.axis_index("c")
        grp_l = sid // 8
        sub = sid % 8
        obufs = (ow0, ow1)
        wsems = (ws0, ws1)
        ivs = (iv0, iv1)
        isems = (is0, is1)

        def unit_src(st, hh):
            return idx_sp.at[st, :, pl.ds(hh * 4, 4), :]

        @pl.when(sid == 0)
        def _stage_idx():
            pltpu.sync_copy(idx_hbm, idx_sp)

        plsc.subcore_barrier()

        for p in range(_PASSES):
            g = 4 * p + 2 * cid + grp_l
            e = 8 * g + sub
            pltpu.sync_copy(tp3_hbm.at[e], row_v)
            pltpu.sync_copy(pe8_hbm.at[e], pe_v)
            pltpu.async_copy(unit_src(0, 0), iv0, is0)
            pltpu.async_copy(unit_src(0, 1), iv1, is1)

            def half(st, hh, first_round):
                iv = ivs[hh]
                isem = isems[hh]
                ob = obufs[hh]
                wsem = wsems[hh]
                dst = out_hbm.at[pl.ds(st * 8 + hh * 4, 4), g, :, sub, :]
                pltpu.make_async_copy(unit_src(st, hh), iv, isem).wait()
                if first_round:
                    @pl.when(st >= 1)
                    def _w():
                        pltpu.make_async_copy(ob, dst, wsem).wait()
                else:
                    pltpu.make_async_copy(ob, dst, wsem).wait()
                zero16 = jnp.zeros((16,), dtype=jnp.int32)
                for si in range(4):
                    s = st * 8 + hh * 4 + si
                    hi16 = jnp.full((16,), s // 128, dtype=jnp.int32)
                    lo16 = jnp.full((16,), s % 128, dtype=jnp.int32)
                    p0 = plsc.load_gather(pe_v, [hi16, lo16])

                    @plsc.parallel_loop(0, 64, 1, unroll=64)
                    def _chunk(k, si=si, p0=p0, ob=ob, iv=iv):
                        tok = iv[k >> 3, si, pl.ds((k & 7) * 16, 16)]
                        vals = plsc.load_gather(row_v, [zero16, tok])
                        ob[si, k >> 3, pl.ds((k & 7) * 16, 16)] = vals + p0
                pltpu.async_copy(ob, dst, wsem)

                @pl.when(st + 1 < _SG)
                def _pf():
                    pltpu.async_copy(unit_src(st + 1, hh), iv, isem)

            def st_body(st, carry):
                half(st, 0, p == 0)
                half(st, 1, p == 0)
                return carry

            lax.fori_loop(0, _SG, st_body, 0)

        dummy = out_hbm.at[pl.ds(0, 4), 0, :, 0, :]
        pltpu.make_async_copy(obufs[0], dummy, wsems[0]).wait()
        pltpu.make_async_copy(obufs[1], dummy, wsems[1]).wait()

    return k(idx_flat, tp3, pe8)


def kernel(input_x, table):
    idx4 = (input_x.astype(jnp.int32)
            .reshape(8, 128, 25, 8).transpose(2, 0, 3, 1))
    tp3 = jnp.pad(table, ((0, _VP - _V), (0, 0))).T.reshape(_E, 782, 128)
    pe8 = _pe_table_t()
    out5 = _sc_call(idx4, tp3, pe8)
    return out5.transpose(2, 4, 0, 1, 3).reshape(_B, _S, _E)
